# baseline (speedup 1.0000x reference)
import functools

import jax
import jax.numpy as jnp
from jax import lax
from jax.experimental import pallas as pl
from jax.experimental.pallas import tpu as pltpu
from jax.experimental.pallas import tpu_sc as plsc

_N = 10000
_E = 320000
_D = 128
_G = 16
_NC = 2
_NS = 16
_NW = _NC * _NS
_L = 16

_CHUNK = 128
_RT = 80
_RTOT = _RT * _NW
_EPAD = _RTOT * _CHUNK

_NP = 10240
_RSUB = _NP // _NS

_M = 1000
_NB = _N // _M
_M1 = 1280
_NB1 = _NP // _M1


def _bcast_lane(v, r):
  idx = jnp.full((_L, 1), r, dtype=jnp.int32)
  return lax.gather(
      v, idx,
      lax.GatherDimensionNumbers(
          offset_dims=(), collapsed_slice_dims=(0,), start_index_map=(0,)),
      (1,),
      mode=lax.GatherScatterMode.PROMISE_IN_BOUNDS)


def _deg_body(w_h, dst_h, out_h, degv, dstv, wv):
  c = lax.axis_index("c")
  s = lax.axis_index("s")
  wid = s * _NC + c
  lo = wid * _RT * _CHUNK

  @pl.loop(0, _NP // _L)
  def _(k):
    degv[pl.ds(k * _L, _L)] = jnp.zeros((_L,), jnp.float32)

  pltpu.sync_copy(dst_h.at[pl.ds(lo, _RT * _CHUNK)], dstv)
  pltpu.sync_copy(w_h.at[pl.ds(lo, _RT * _CHUNK)], wv)

  @pl.loop(0, _RT * _CHUNK // _L)
  def _(k):
    d16 = dstv[pl.ds(k * _L, _L)]
    w16 = wv[pl.ds(k * _L, _L)]
    plsc.addupdate_scatter(degv, [d16], w16)

  pltpu.sync_copy(degv, out_h.at[wid])


@functools.cache
def _get_deg():
  return pl.kernel(
      _deg_body,
      out_type=jax.ShapeDtypeStruct((_NW, _NP), jnp.float32),
      mesh=plsc.VectorSubcoreMesh(core_axis_name="c", subcore_axis_name="s",
                                  num_cores=_NC, num_subcores=_NS),
      scratch_types=[
          pltpu.VMEM((_NP,), jnp.float32),
          pltpu.VMEM((_RT * _CHUNK,), jnp.int32),
          pltpu.VMEM((_RT * _CHUNK,), jnp.float32),
      ],
      compiler_params=pltpu.CompilerParams(needs_layout_passes=False),
  )


_SB = 2
_NSB = _RT // _SB


def _scale_chunk(buf, wb, wbase):
  for i in range(_CHUNK // _L):
    wvec = wb[pl.ds(wbase + i * _L, _L)]
    for r in range(_L):
      e = i * _L + r
      b = _bcast_lane(wvec, r)
      for f in range(0, _D, _L):
        buf[e, pl.ds(f, _L)] = buf[e, pl.ds(f, _L)] * b


def _spread_body(y_h, src_h, dst_h, w_h, z_h, out_h,
                 acc_sh, sidx, didx, wb,
                 rows0, rows1, gsem0, gsem1, ssem0, ssem1):
  c = lax.axis_index("c")
  s = lax.axis_index("s")
  wid = s * _NC + c
  r0 = s * _RSUB
  pltpu.sync_copy(z_h.at[pl.ds(r0, _RSUB)], acc_sh.at[pl.ds(r0, _RSUB)])
  plsc.subcore_barrier()
  lo = wid * _RT

  rows = (rows0, rows1)
  gsem = (gsem0, gsem1)
  ssem = (ssem0, ssem1)

  def stage(sb_dyn, half):
    pltpu.sync_copy(src_h.at[pl.ds(lo + sb_dyn * _SB, _SB)],
                    sidx.at[pl.ds(half * _SB, _SB)])
    pltpu.sync_copy(dst_h.at[pl.ds(lo + sb_dyn * _SB, _SB)],
                    didx.at[pl.ds(half * _SB, _SB)])
    pltpu.sync_copy(w_h.at[pl.ds((lo + sb_dyn * _SB) * _CHUNK, _SB * _CHUNK)],
                    wb.at[pl.ds(half * _SB * _CHUNK, _SB * _CHUNK)])

  def wait_gather(idx_ref, bb):
    pltpu.make_async_copy(y_h.at[idx_ref], rows[bb], gsem[bb]).wait()

  def wait_scatter(bb):
    pltpu.make_async_copy(rows[bb], acc_sh.at[didx.at[0]], ssem[bb]).wait()

  def issue_scatter(slot, bb):
    pltpu.async_copy(rows[bb], acc_sh.at[didx.at[slot]], ssem[bb], add=True)

  def prefetch(idx_ref, bb):
    pltpu.async_copy(y_h.at[idx_ref], rows[bb], gsem[bb])

  stage(0, 0)
  prefetch(sidx.at[0], 0)
  stage(1, 1)
  for t in range(_SB):
    bb = t % 2
    wait_gather(sidx.at[t], bb)
    if t >= 1:
      wait_scatter(1 - bb)
    nslot = t + 1 if t + 1 < _SB else _SB
    prefetch(sidx.at[nslot], 1 - bb)
    _scale_chunk(rows[bb], wb, t * _CHUNK)
    issue_scatter(t, bb)

  @pl.loop(1, _NSB)
  def _(sb):
    sbp = jnp.bitwise_and(sb, 1)
    nsbp = 1 - sbp
    sbn = jnp.minimum(sb + 1, _NSB - 1)
    stage(sbn, nsbp)
    for t in range(_SB):
      bb = t % 2
      wait_gather(sidx.at[sbp * _SB + t], bb)
      wait_scatter(1 - bb)
      if t + 1 < _SB:
        nref = sidx.at[sbp * _SB + t + 1]
      else:
        nref = sidx.at[nsbp * _SB]
      prefetch(nref, 1 - bb)
      _scale_chunk(rows[bb], wb, sbp * (_SB * _CHUNK) + t * _CHUNK)
      issue_scatter(sbp * _SB + t, bb)

  wait_scatter(1)
  pltpu.make_async_copy(y_h.at[sidx.at[0]], rows0, gsem0).wait()
  plsc.subcore_barrier()
  pltpu.sync_copy(acc_sh.at[pl.ds(r0, _RSUB)], out_h.at[c, pl.ds(r0, _RSUB)])


@functools.cache
def _get_spread():
  return pl.kernel(
      _spread_body,
      out_type=jax.ShapeDtypeStruct((_NC, _NP, _D), jnp.float32),
      mesh=plsc.VectorSubcoreMesh(core_axis_name="c", subcore_axis_name="s",
                                  num_cores=_NC, num_subcores=_NS),
      scratch_types=[
          pltpu.VMEM_SHARED((_NP, _D), jnp.float32),
          pltpu.VMEM((2 * _SB, _CHUNK), jnp.int32),
          pltpu.VMEM((2 * _SB, _CHUNK), jnp.int32),
          pltpu.VMEM((2 * _SB * _CHUNK,), jnp.float32),
          pltpu.VMEM((_CHUNK, _D), jnp.float32),
          pltpu.VMEM((_CHUNK, _D), jnp.float32),
          pltpu.SemaphoreType.DMA,
          pltpu.SemaphoreType.DMA,
          pltpu.SemaphoreType.DMA,
          pltpu.SemaphoreType.DMA,
      ],
  )


def _mm1_body(degp_ref, x_ref, w_ref, y_ref, dis_ref):
  i = pl.program_id(0)
  deg = jnp.sum(degp_ref[:, pl.ds(i * _M1, _M1)], axis=0)[:, None] + 1.0
  dis = lax.rsqrt(deg)
  dis_ref[...] = dis
  y_ref[...] = jnp.dot(x_ref[...], w_ref[...],
                       preferred_element_type=jnp.float32) * dis


_mm1 = pl.pallas_call(
    _mm1_body,
    grid=(_NB1,),
    in_specs=[
        pl.BlockSpec((_NW, _NP), lambda i: (0, 0)),
        pl.BlockSpec((_M1, _D), lambda i: (i, 0)),
        pl.BlockSpec((_D, _D), lambda i: (0, 0)),
    ],
    out_specs=[
        pl.BlockSpec((_M1, _D), lambda i: (i, 0)),
        pl.BlockSpec((_M1, 1), lambda i: (i, 0)),
    ],
    out_shape=[
        jax.ShapeDtypeStruct((_N, _D), jnp.float32),
        jax.ShapeDtypeStruct((_N, 1), jnp.float32),
    ],
)


def _mm2_body(p_ref, y1_ref, dis_ref, b_ref, w2_ref, y2_ref):
  h = dis_ref[...] * (p_ref[0] + p_ref[1] + y1_ref[...]) + b_ref[...]
  h = jnp.maximum(h, 0.0)
  y2_ref[...] = jnp.dot(h, w2_ref[...],
                        preferred_element_type=jnp.float32) * dis_ref[...]


_mm2 = pl.pallas_call(
    _mm2_body,
    grid=(_NB,),
    in_specs=[
        pl.BlockSpec((_NC, _M, _D), lambda i: (0, i, 0)),
        pl.BlockSpec((_M, _D), lambda i: (i, 0)),
        pl.BlockSpec((_M, 1), lambda i: (i, 0)),
        pl.BlockSpec((1, _D), lambda i: (0, 0)),
        pl.BlockSpec((_D, _D), lambda i: (0, 0)),
    ],
    out_specs=pl.BlockSpec((_M, _D), lambda i: (i, 0)),
    out_shape=jax.ShapeDtypeStruct((_N, _D), jnp.float32),
)


def _pool_body(p_ref, y2_ref, dis_ref, b_ref, batch_ref, out_ref, acc, cnt):
  i = pl.program_id(0)
  h = dis_ref[...] * (p_ref[0] + p_ref[1] + y2_ref[...]) + b_ref[...]
  h = jnp.maximum(h, 0.0)
  bvec = batch_ref[0, 0, :]
  gids = lax.broadcasted_iota(jnp.int32, (_G, _M), 0)
  oh = (bvec[None, :] == gids).astype(jnp.float32)
  part = jnp.dot(oh, h, preferred_element_type=jnp.float32)
  cpart = jnp.sum(oh, axis=1, keepdims=True)

  @pl.when(i == 0)
  def _():
    acc[...] = part
    cnt[...] = cpart

  @pl.when(i > 0)
  def _():
    acc[...] += part
    cnt[...] += cpart

  @pl.when(i == _NB - 1)
  def _():
    out_ref[...] = acc[...] / jnp.maximum(cnt[...], 1.0)


_pool = pl.pallas_call(
    _pool_body,
    grid=(_NB,),
    in_specs=[
        pl.BlockSpec((_NC, _M, _D), lambda i: (0, i, 0)),
        pl.BlockSpec((_M, _D), lambda i: (i, 0)),
        pl.BlockSpec((_M, 1), lambda i: (i, 0)),
        pl.BlockSpec((1, _D), lambda i: (0, 0)),
        pl.BlockSpec((1, 1, _M), lambda i: (i, 0, 0)),
    ],
    out_specs=pl.BlockSpec((_G, _D), lambda i: (0, 0)),
    out_shape=jax.ShapeDtypeStruct((_G, _D), jnp.float32),
    scratch_shapes=[
        pltpu.VMEM((_G, _D), jnp.float32),
        pltpu.VMEM((_G, 1), jnp.float32),
    ],
)


def kernel(x, edge_index, batch, edge_weight, W1, b1, W2, b2):
  pad = _EPAD - _E
  src = jnp.concatenate(
      [edge_index[0], jnp.zeros((pad,), jnp.int32)]).reshape(_RTOT, _CHUNK)
  dstf = jnp.concatenate([edge_index[1], jnp.zeros((pad,), jnp.int32)])
  dst = dstf.reshape(_RTOT, _CHUNK)
  w = jnp.concatenate([edge_weight, jnp.zeros((pad,), jnp.float32)])
  z128 = jnp.zeros((_NP, _D), jnp.float32)
  batch3 = batch.reshape(_NB, 1, _M)

  deg_k = _get_deg()
  spread_k = _get_spread()
  degp = deg_k(w, dstf)
  y1, dis = _mm1(degp, x, W1)
  p1 = spread_k(y1, src, dst, w, z128)
  y2 = _mm2(p1, y1, dis, b1.reshape(1, _D), W2)
  p2 = spread_k(y2, src, dst, w, z128)
  return _pool(p2, y2, dis, b2.reshape(1, _D), batch3)


# speedup vs baseline: 1.0020x; 1.0020x over previous
import functools

import jax
import jax.numpy as jnp
from jax import lax
from jax.experimental import pallas as pl
from jax.experimental.pallas import tpu as pltpu
from jax.experimental.pallas import tpu_sc as plsc

_N = 10000
_E = 320000
_D = 128
_G = 16
_NC = 2
_NS = 16
_NW = _NC * _NS
_L = 16

_CHUNK = 128
_RT = 80
_RTOT = _RT * _NW
_EPAD = _RTOT * _CHUNK

_NP = 10240
_RSUB = _NP // _NS

_M = 1000
_NB = _N // _M
_M1 = 1280
_NB1 = _NP // _M1


def _bcast_lane(v, r):
  idx = jnp.full((_L, 1), r, dtype=jnp.int32)
  return lax.gather(
      v, idx,
      lax.GatherDimensionNumbers(
          offset_dims=(), collapsed_slice_dims=(0,), start_index_map=(0,)),
      (1,),
      mode=lax.GatherScatterMode.PROMISE_IN_BOUNDS)


def _deg_body(w_h, dst_h, out_h, degv, dstv, wv):
  c = lax.axis_index("c")
  s = lax.axis_index("s")
  wid = s * _NC + c
  lo = wid * _RT * _CHUNK

  @pl.loop(0, _NP // _L)
  def _(k):
    degv[pl.ds(k * _L, _L)] = jnp.zeros((_L,), jnp.float32)

  pltpu.sync_copy(dst_h.at[pl.ds(lo, _RT * _CHUNK)], dstv)
  pltpu.sync_copy(w_h.at[pl.ds(lo, _RT * _CHUNK)], wv)

  @pl.loop(0, _RT * _CHUNK // _L)
  def _(k):
    d16 = dstv[pl.ds(k * _L, _L)]
    w16 = wv[pl.ds(k * _L, _L)]
    plsc.addupdate_scatter(degv, [d16], w16)

  pltpu.sync_copy(degv, out_h.at[wid])


@functools.cache
def _get_deg():
  return pl.kernel(
      _deg_body,
      out_type=jax.ShapeDtypeStruct((_NW, _NP), jnp.float32),
      mesh=plsc.VectorSubcoreMesh(core_axis_name="c", subcore_axis_name="s",
                                  num_cores=_NC, num_subcores=_NS),
      scratch_types=[
          pltpu.VMEM((_NP,), jnp.float32),
          pltpu.VMEM((_RT * _CHUNK,), jnp.int32),
          pltpu.VMEM((_RT * _CHUNK,), jnp.float32),
      ],
      compiler_params=pltpu.CompilerParams(needs_layout_passes=False),
  )


_SB = 4
_NSB = _RT // _SB


def _scale_chunk(buf, wb, wbase):
  for i in range(_CHUNK // _L):
    wvec = wb[pl.ds(wbase + i * _L, _L)]
    for r in range(_L):
      e = i * _L + r
      bc = _bcast_lane(wvec, r)
      for f in range(0, _D, _L):
        buf[e, pl.ds(f, _L)] = buf[e, pl.ds(f, _L)] * bc


def _spread_body(y_h, src_h, dst_h, w_h, z_h, out_h,
                 acc_sh, sidx, didx, wb, rows0, rows1, gsem0, gsem1):
  c = lax.axis_index("c")
  s = lax.axis_index("s")
  wid = s * _NC + c
  r0 = s * _RSUB
  pltpu.sync_copy(z_h.at[pl.ds(r0, _RSUB)], acc_sh.at[pl.ds(r0, _RSUB)])
  plsc.subcore_barrier()
  lo = wid * _RT

  rows = (rows0, rows1)
  gsem = (gsem0, gsem1)

  def stage(sb_dyn, half):
    pltpu.sync_copy(src_h.at[pl.ds(lo + sb_dyn * _SB, _SB)],
                    sidx.at[pl.ds(half * _SB, _SB)])
    pltpu.sync_copy(dst_h.at[pl.ds(lo + sb_dyn * _SB, _SB)],
                    didx.at[pl.ds(half * _SB, _SB)])
    pltpu.sync_copy(w_h.at[pl.ds((lo + sb_dyn * _SB) * _CHUNK, _SB * _CHUNK)],
                    wb.at[pl.ds(half * _SB * _CHUNK, _SB * _CHUNK)])

  stage(0, 0)
  pltpu.async_copy(y_h.at[sidx.at[0]], rows0, gsem0)

  @pl.loop(0, _NSB)
  def _(sb):
    sbp = jnp.bitwise_and(sb, 1)
    nsbp = 1 - sbp
    sbn = jnp.minimum(sb + 1, _NSB - 1)
    stage(sbn, nsbp)
    for t in range(_SB):
      bb = t % 2
      pltpu.make_async_copy(y_h.at[sidx.at[sbp * _SB + t]],
                            rows[bb], gsem[bb]).wait()
      if t + 1 < _SB:
        nref = sidx.at[sbp * _SB + t + 1]
      else:
        nref = sidx.at[nsbp * _SB]
      pltpu.async_copy(y_h.at[nref], rows[1 - bb], gsem[1 - bb])
      _scale_chunk(rows[bb], wb, sbp * (_SB * _CHUNK) + t * _CHUNK)
      pltpu.sync_copy(rows[bb], acc_sh.at[didx.at[sbp * _SB + t]], add=True)

  pltpu.make_async_copy(y_h.at[sidx.at[0]], rows0, gsem0).wait()
  plsc.subcore_barrier()
  pltpu.sync_copy(acc_sh.at[pl.ds(r0, _RSUB)], out_h.at[c, pl.ds(r0, _RSUB)])


@functools.cache
def _get_spread():
  return pl.kernel(
      _spread_body,
      out_type=jax.ShapeDtypeStruct((_NC, _NP, _D), jnp.float32),
      mesh=plsc.VectorSubcoreMesh(core_axis_name="c", subcore_axis_name="s",
                                  num_cores=_NC, num_subcores=_NS),
      scratch_types=[
          pltpu.VMEM_SHARED((_NP, _D), jnp.float32),
          pltpu.VMEM((2 * _SB, _CHUNK), jnp.int32),
          pltpu.VMEM((2 * _SB, _CHUNK), jnp.int32),
          pltpu.VMEM((2 * _SB * _CHUNK,), jnp.float32),
          pltpu.VMEM((_CHUNK, _D), jnp.float32),
          pltpu.VMEM((_CHUNK, _D), jnp.float32),
          pltpu.SemaphoreType.DMA,
          pltpu.SemaphoreType.DMA,
      ],
  )


def _mm1_body(degp_ref, x_ref, w_ref, y_ref, dis_ref):
  i = pl.program_id(0)
  deg = jnp.sum(degp_ref[:, pl.ds(i * _M1, _M1)], axis=0)[:, None] + 1.0
  dis = lax.rsqrt(deg)
  dis_ref[...] = dis
  y_ref[...] = jnp.dot(x_ref[...], w_ref[...],
                       preferred_element_type=jnp.float32) * dis


_mm1 = pl.pallas_call(
    _mm1_body,
    grid=(_NB1,),
    in_specs=[
        pl.BlockSpec((_NW, _NP), lambda i: (0, 0)),
        pl.BlockSpec((_M1, _D), lambda i: (i, 0)),
        pl.BlockSpec((_D, _D), lambda i: (0, 0)),
    ],
    out_specs=[
        pl.BlockSpec((_M1, _D), lambda i: (i, 0)),
        pl.BlockSpec((_M1, 1), lambda i: (i, 0)),
    ],
    out_shape=[
        jax.ShapeDtypeStruct((_N, _D), jnp.float32),
        jax.ShapeDtypeStruct((_N, 1), jnp.float32),
    ],
)


def _mm2_body(p_ref, y1_ref, dis_ref, b_ref, w2_ref, y2_ref):
  h = dis_ref[...] * (p_ref[0] + p_ref[1] + y1_ref[...]) + b_ref[...]
  h = jnp.maximum(h, 0.0)
  y2_ref[...] = jnp.dot(h, w2_ref[...],
                        preferred_element_type=jnp.float32) * dis_ref[...]


_mm2 = pl.pallas_call(
    _mm2_body,
    grid=(_NB,),
    in_specs=[
        pl.BlockSpec((_NC, _M, _D), lambda i: (0, i, 0)),
        pl.BlockSpec((_M, _D), lambda i: (i, 0)),
        pl.BlockSpec((_M, 1), lambda i: (i, 0)),
        pl.BlockSpec((1, _D), lambda i: (0, 0)),
        pl.BlockSpec((_D, _D), lambda i: (0, 0)),
    ],
    out_specs=pl.BlockSpec((_M, _D), lambda i: (i, 0)),
    out_shape=jax.ShapeDtypeStruct((_N, _D), jnp.float32),
)


def _pool_body(p_ref, y2_ref, dis_ref, b_ref, batch_ref, out_ref, acc, cnt):
  i = pl.program_id(0)
  h = dis_ref[...] * (p_ref[0] + p_ref[1] + y2_ref[...]) + b_ref[...]
  h = jnp.maximum(h, 0.0)
  bvec = batch_ref[0, 0, :]
  gids = lax.broadcasted_iota(jnp.int32, (_G, _M), 0)
  oh = (bvec[None, :] == gids).astype(jnp.float32)
  part = jnp.dot(oh, h, preferred_element_type=jnp.float32)
  cpart = jnp.sum(oh, axis=1, keepdims=True)

  @pl.when(i == 0)
  def _():
    acc[...] = part
    cnt[...] = cpart

  @pl.when(i > 0)
  def _():
    acc[...] += part
    cnt[...] += cpart

  @pl.when(i == _NB - 1)
  def _():
    out_ref[...] = acc[...] / jnp.maximum(cnt[...], 1.0)


_pool = pl.pallas_call(
    _pool_body,
    grid=(_NB,),
    in_specs=[
        pl.BlockSpec((_NC, _M, _D), lambda i: (0, i, 0)),
        pl.BlockSpec((_M, _D), lambda i: (i, 0)),
        pl.BlockSpec((_M, 1), lambda i: (i, 0)),
        pl.BlockSpec((1, _D), lambda i: (0, 0)),
        pl.BlockSpec((1, 1, _M), lambda i: (i, 0, 0)),
    ],
    out_specs=pl.BlockSpec((_G, _D), lambda i: (0, 0)),
    out_shape=jax.ShapeDtypeStruct((_G, _D), jnp.float32),
    scratch_shapes=[
        pltpu.VMEM((_G, _D), jnp.float32),
        pltpu.VMEM((_G, 1), jnp.float32),
    ],
)


def kernel(x, edge_index, batch, edge_weight, W1, b1, W2, b2):
  pad = _EPAD - _E
  src = jnp.concatenate(
      [edge_index[0], jnp.zeros((pad,), jnp.int32)]).reshape(_RTOT, _CHUNK)
  dstf = jnp.concatenate([edge_index[1], jnp.zeros((pad,), jnp.int32)])
  dst = dstf.reshape(_RTOT, _CHUNK)
  w = jnp.concatenate([edge_weight, jnp.zeros((pad,), jnp.float32)])
  z128 = jnp.zeros((_NP, _D), jnp.float32)
  batch3 = batch.reshape(_NB, 1, _M)

  deg_k = _get_deg()
  spread_k = _get_spread()
  degp = deg_k(w, dstf)
  y1, dis = _mm1(degp, x, W1)
  p1 = spread_k(y1, src, dst, w, z128)
  y2 = _mm2(p1, y1, dis, b1.reshape(1, _D), W2)
  p2 = spread_k(y2, src, dst, w, z128)
  return _pool(p2, y2, dis, b2.reshape(1, _D), batch3)


# speedup vs baseline: 1.1554x; 1.1532x over previous
import functools

import jax
import jax.numpy as jnp
from jax import lax
from jax.experimental import pallas as pl
from jax.experimental.pallas import tpu as pltpu
from jax.experimental.pallas import tpu_sc as plsc

_N = 10000
_E = 320000
_D = 128
_G = 16
_NC = 2
_NS = 16
_NW = _NC * _NS
_L = 16

_CHUNK = 128
_RT = 80
_RTOT = _RT * _NW
_EPAD = _RTOT * _CHUNK

_NP = 10240
_RSUB = _NP // _NS

_M = 1000
_NB = _N // _M
_M1 = 1280
_NB1 = _NP // _M1


def _bcast_lane(v, r):
  idx = jnp.full((_L, 1), r, dtype=jnp.int32)
  return lax.gather(
      v, idx,
      lax.GatherDimensionNumbers(
          offset_dims=(), collapsed_slice_dims=(0,), start_index_map=(0,)),
      (1,),
      mode=lax.GatherScatterMode.PROMISE_IN_BOUNDS)


def _deg_body(w_h, dst_h, out_h, degv, dstv, wv):
  c = lax.axis_index("c")
  s = lax.axis_index("s")
  wid = s * _NC + c
  lo = wid * _RT * _CHUNK

  @pl.loop(0, _NP // _L)
  def _(k):
    degv[pl.ds(k * _L, _L)] = jnp.zeros((_L,), jnp.float32)

  pltpu.sync_copy(dst_h.at[pl.ds(lo, _RT * _CHUNK)], dstv)
  pltpu.sync_copy(w_h.at[pl.ds(lo, _RT * _CHUNK)], wv)

  @pl.loop(0, _RT * _CHUNK // _L)
  def _(k):
    d16 = dstv[pl.ds(k * _L, _L)]
    w16 = wv[pl.ds(k * _L, _L)]
    plsc.addupdate_scatter(degv, [d16], w16)

  pltpu.sync_copy(degv, out_h.at[wid])


@functools.cache
def _get_deg():
  return pl.kernel(
      _deg_body,
      out_type=jax.ShapeDtypeStruct((_NW, _NP), jnp.float32),
      mesh=plsc.VectorSubcoreMesh(core_axis_name="c", subcore_axis_name="s",
                                  num_cores=_NC, num_subcores=_NS),
      scratch_types=[
          pltpu.VMEM((_NP,), jnp.float32),
          pltpu.VMEM((_RT * _CHUNK,), jnp.int32),
          pltpu.VMEM((_RT * _CHUNK,), jnp.float32),
      ],
      compiler_params=pltpu.CompilerParams(needs_layout_passes=False),
  )


_SB = 4
_NSB = _RT // _SB
_RTA = 112
_RTB = 48
_RT2 = _RTA + _RTB


def _scale_chunk(buf, wb, wbase):
  for i in range(_CHUNK // _L):
    wvec = wb[pl.ds(wbase + i * _L, _L)]
    for r in range(_L):
      e = i * _L + r
      bc = _bcast_lane(wvec, r)
      for f in range(0, _D, _L):
        buf[e, pl.ds(f, _L)] = buf[e, pl.ds(f, _L)] * bc


def _spread_body(y_h, src_h, dst_h, w_h, z_h, out_h,
                 acc_sh, sidx, didx, wb, rows0, rows1, gsem0, gsem1):
  c = lax.axis_index("c")
  s = lax.axis_index("s")
  wid = s * _NC + c
  r0 = s * _RSUB
  pltpu.sync_copy(z_h.at[pl.ds(r0, _RSUB)], acc_sh.at[pl.ds(r0, _RSUB)])
  plsc.subcore_barrier()
  lo = s * _RT2 + c * _RTA
  nsb_c = jnp.where(c == 0, _RTA // _SB, _RTB // _SB)

  rows = (rows0, rows1)
  gsem = (gsem0, gsem1)

  def stage(sb_dyn, half):
    pltpu.sync_copy(src_h.at[pl.ds(lo + sb_dyn * _SB, _SB)],
                    sidx.at[pl.ds(half * _SB, _SB)])
    pltpu.sync_copy(dst_h.at[pl.ds(lo + sb_dyn * _SB, _SB)],
                    didx.at[pl.ds(half * _SB, _SB)])
    pltpu.sync_copy(w_h.at[pl.ds((lo + sb_dyn * _SB) * _CHUNK, _SB * _CHUNK)],
                    wb.at[pl.ds(half * _SB * _CHUNK, _SB * _CHUNK)])

  stage(0, 0)
  pltpu.async_copy(y_h.at[sidx.at[0]], rows0, gsem0)

  @pl.loop(0, nsb_c)
  def _(sb):
    sbp = jnp.bitwise_and(sb, 1)
    nsbp = 1 - sbp
    sbn = jnp.minimum(sb + 1, nsb_c - 1)
    stage(sbn, nsbp)
    for t in range(_SB):
      bb = t % 2
      pltpu.make_async_copy(y_h.at[sidx.at[sbp * _SB + t]],
                            rows[bb], gsem[bb]).wait()
      if t + 1 < _SB:
        nref = sidx.at[sbp * _SB + t + 1]
      else:
        nref = sidx.at[nsbp * _SB]
      pltpu.async_copy(y_h.at[nref], rows[1 - bb], gsem[1 - bb])
      _scale_chunk(rows[bb], wb, sbp * (_SB * _CHUNK) + t * _CHUNK)
      pltpu.sync_copy(rows[bb], acc_sh.at[didx.at[sbp * _SB + t]], add=True)

  pltpu.make_async_copy(y_h.at[sidx.at[0]], rows0, gsem0).wait()
  plsc.subcore_barrier()
  pltpu.sync_copy(acc_sh.at[pl.ds(r0, _RSUB)], out_h.at[c, pl.ds(r0, _RSUB)])


@functools.cache
def _get_spread():
  return pl.kernel(
      _spread_body,
      out_type=jax.ShapeDtypeStruct((_NC, _NP, _D), jnp.float32),
      mesh=plsc.VectorSubcoreMesh(core_axis_name="c", subcore_axis_name="s",
                                  num_cores=_NC, num_subcores=_NS),
      scratch_types=[
          pltpu.VMEM_SHARED((_NP, _D), jnp.float32),
          pltpu.VMEM((2 * _SB, _CHUNK), jnp.int32),
          pltpu.VMEM((2 * _SB, _CHUNK), jnp.int32),
          pltpu.VMEM((2 * _SB * _CHUNK,), jnp.float32),
          pltpu.VMEM((_CHUNK, _D), jnp.float32),
          pltpu.VMEM((_CHUNK, _D), jnp.float32),
          pltpu.SemaphoreType.DMA,
          pltpu.SemaphoreType.DMA,
      ],
  )


def _mm1_body(degp_ref, x_ref, w_ref, y_ref, dis_ref):
  i = pl.program_id(0)
  deg = jnp.sum(degp_ref[:, pl.ds(i * _M1, _M1)], axis=0)[:, None] + 1.0
  dis = lax.rsqrt(deg)
  dis_ref[...] = dis
  y_ref[...] = jnp.dot(x_ref[...], w_ref[...],
                       preferred_element_type=jnp.float32) * dis


_mm1 = pl.pallas_call(
    _mm1_body,
    grid=(_NB1,),
    in_specs=[
        pl.BlockSpec((_NW, _NP), lambda i: (0, 0)),
        pl.BlockSpec((_M1, _D), lambda i: (i, 0)),
        pl.BlockSpec((_D, _D), lambda i: (0, 0)),
    ],
    out_specs=[
        pl.BlockSpec((_M1, _D), lambda i: (i, 0)),
        pl.BlockSpec((_M1, 1), lambda i: (i, 0)),
    ],
    out_shape=[
        jax.ShapeDtypeStruct((_N, _D), jnp.float32),
        jax.ShapeDtypeStruct((_N, 1), jnp.float32),
    ],
)


def _mm2_body(p_ref, y1_ref, dis_ref, b_ref, w2_ref, y2_ref):
  h = dis_ref[...] * (p_ref[0] + p_ref[1] + y1_ref[...]) + b_ref[...]
  h = jnp.maximum(h, 0.0)
  y2_ref[...] = jnp.dot(h, w2_ref[...],
                        preferred_element_type=jnp.float32) * dis_ref[...]


_mm2 = pl.pallas_call(
    _mm2_body,
    grid=(_NB,),
    in_specs=[
        pl.BlockSpec((_NC, _M, _D), lambda i: (0, i, 0)),
        pl.BlockSpec((_M, _D), lambda i: (i, 0)),
        pl.BlockSpec((_M, 1), lambda i: (i, 0)),
        pl.BlockSpec((1, _D), lambda i: (0, 0)),
        pl.BlockSpec((_D, _D), lambda i: (0, 0)),
    ],
    out_specs=pl.BlockSpec((_M, _D), lambda i: (i, 0)),
    out_shape=jax.ShapeDtypeStruct((_N, _D), jnp.float32),
)


def _pool_body(p_ref, y2_ref, dis_ref, b_ref, batch_ref, out_ref, acc, cnt):
  i = pl.program_id(0)
  h = dis_ref[...] * (p_ref[0] + p_ref[1] + y2_ref[...]) + b_ref[...]
  h = jnp.maximum(h, 0.0)
  bvec = batch_ref[0, 0, :]
  gids = lax.broadcasted_iota(jnp.int32, (_G, _M), 0)
  oh = (bvec[None, :] == gids).astype(jnp.float32)
  part = jnp.dot(oh, h, preferred_element_type=jnp.float32)
  cpart = jnp.sum(oh, axis=1, keepdims=True)

  @pl.when(i == 0)
  def _():
    acc[...] = part
    cnt[...] = cpart

  @pl.when(i > 0)
  def _():
    acc[...] += part
    cnt[...] += cpart

  @pl.when(i == _NB - 1)
  def _():
    out_ref[...] = acc[...] / jnp.maximum(cnt[...], 1.0)


_pool = pl.pallas_call(
    _pool_body,
    grid=(_NB,),
    in_specs=[
        pl.BlockSpec((_NC, _M, _D), lambda i: (0, i, 0)),
        pl.BlockSpec((_M, _D), lambda i: (i, 0)),
        pl.BlockSpec((_M, 1), lambda i: (i, 0)),
        pl.BlockSpec((1, _D), lambda i: (0, 0)),
        pl.BlockSpec((1, 1, _M), lambda i: (i, 0, 0)),
    ],
    out_specs=pl.BlockSpec((_G, _D), lambda i: (0, 0)),
    out_shape=jax.ShapeDtypeStruct((_G, _D), jnp.float32),
    scratch_shapes=[
        pltpu.VMEM((_G, _D), jnp.float32),
        pltpu.VMEM((_G, 1), jnp.float32),
    ],
)


def kernel(x, edge_index, batch, edge_weight, W1, b1, W2, b2):
  pad = _EPAD - _E
  src = jnp.concatenate(
      [edge_index[0], jnp.zeros((pad,), jnp.int32)]).reshape(_RTOT, _CHUNK)
  dstf = jnp.concatenate([edge_index[1], jnp.zeros((pad,), jnp.int32)])
  dst = dstf.reshape(_RTOT, _CHUNK)
  w = jnp.concatenate([edge_weight, jnp.zeros((pad,), jnp.float32)])
  z128 = jnp.zeros((_NP, _D), jnp.float32)
  batch3 = batch.reshape(_NB, 1, _M)

  deg_k = _get_deg()
  spread_k = _get_spread()
  degp = deg_k(w, dstf)
  y1, dis = _mm1(degp, x, W1)
  p1 = spread_k(y1, src, dst, w, z128)
  y2 = _mm2(p1, y1, dis, b1.reshape(1, _D), W2)
  p2 = spread_k(y2, src, dst, w, z128)
  return _pool(p2, y2, dis, b2.reshape(1, _D), batch3)
